# baseline (speedup 1.0000x reference)
import functools

import jax
import jax.numpy as jnp
from jax import lax
from jax.experimental import pallas as pl
from jax.experimental.pallas import tpu as pltpu
from jax.experimental.pallas import tpu_sc as plsc

M, N, DIM = 128, 128, 256
ALPHA = 0.3
SIGMA = max(M, N) / 2.0
NUM_EPOCHS = 100

ROWS = M * N
NC, NS, NLANE = 2, 16, 16
NW = NC * NS
RPW = ROWS // NW
NCH = DIM // NLANE
CH1 = 256
CH2 = 128


def _mesh():
    return plsc.VectorSubcoreMesh(core_axis_name="c", subcore_axis_name="s")


@functools.partial(
    pl.kernel,
    mesh=_mesh(),
    compiler_params=pltpu.CompilerParams(needs_layout_passes=False),
    out_type=[
        jax.ShapeDtypeStruct((NW, NLANE), jnp.float32),
        jax.ShapeDtypeStruct((NW, NLANE), jnp.int32),
    ],
    scratch_types=[
        pltpu.VMEM((DIM,), jnp.float32),
        pltpu.VMEM((CH1, DIM), jnp.float32),
        pltpu.VMEM((NLANE,), jnp.float32),
        pltpu.VMEM((NLANE,), jnp.int32),
    ],
)
def _phase1(w_hbm, x_hbm, dist_out, idx_out, xv, buf, sd, si):
    wid = lax.axis_index("s") * NC + lax.axis_index("c")
    base = wid * RPW
    pltpu.sync_copy(x_hbm, xv)
    xs = [xv[pl.ds(c * NLANE, NLANE)] for c in range(NCH)]
    best = (jnp.float32(jnp.inf), jnp.int32(0))
    for k in range(RPW // CH1):
        pltpu.sync_copy(w_hbm.at[pl.ds(base + k * CH1, CH1)], buf)

        def row_body(r, carry, _k=k):
            bd, bi = carry
            acc = jnp.zeros((NLANE,), jnp.float32)
            for c in range(NCH):
                d = buf[r, pl.ds(c * NLANE, NLANE)] - xs[c]
                acc = acc + d * d
            s = plsc.cumsum(acc)[NLANE - 1]
            better = s < bd
            gi = base + _k * CH1 + r
            return (jnp.where(better, s, bd), jnp.where(better, gi, bi))

        best = lax.fori_loop(0, CH1, row_body, best)
    sd[...] = jnp.full((NLANE,), best[0], jnp.float32)
    si[...] = jnp.full((NLANE,), best[1], jnp.int32)
    pltpu.sync_copy(sd, dist_out.at[wid])
    pltpu.sync_copy(si, idx_out.at[wid])


@functools.partial(
    pl.kernel,
    mesh=_mesh(),
    compiler_params=pltpu.CompilerParams(needs_layout_passes=False),
    out_type=[
        jax.ShapeDtypeStruct((ROWS, DIM), jnp.float32),
        jax.ShapeDtypeStruct((NLANE,), jnp.int32),
    ],
    scratch_types=[
        pltpu.VMEM((DIM,), jnp.float32),
        pltpu.VMEM((NLANE,), jnp.float32),
        pltpu.VMEM((NW, NLANE), jnp.float32),
        pltpu.VMEM((NW, NLANE), jnp.int32),
        pltpu.VMEM((CH2, DIM), jnp.float32),
        pltpu.VMEM((CH2, DIM), jnp.float32),
        pltpu.VMEM((CH2,), jnp.float32),
        pltpu.VMEM((NLANE,), jnp.int32),
    ],
)
def _phase2(w_hbm, x_hbm, par_hbm, dist_hbm, idx_hbm, out_hbm, bmu_out,
            xv, pv, cd, ci, bin_, bout, ratev, bstage):
    wid = lax.axis_index("s") * NC + lax.axis_index("c")
    base = wid * RPW
    pltpu.sync_copy(x_hbm, xv)
    pltpu.sync_copy(par_hbm, pv)
    pltpu.sync_copy(dist_hbm, cd)
    pltpu.sync_copy(idx_hbm, ci)
    pvv = pv[...]
    alpha_op = pvv[0]
    inv_sig2 = pvv[1]
    bd = cd[0][0]
    bi = ci[0][0]
    for w in range(1, NW):
        dw = cd[w][0]
        iw = ci[w][0]
        better = dw < bd
        bd = jnp.where(better, dw, bd)
        bi = jnp.where(better, iw, bi)
    bmu_i = lax.shift_right_logical(bi, 7)
    bmu_j = lax.bitwise_and(bi, jnp.int32(N - 1))
    lane = lax.iota(jnp.int32, NLANE)
    xs = [xv[pl.ds(c * NLANE, NLANE)] for c in range(NCH)]
    for k in range(RPW // CH2):
        cbase = base + k * CH2
        pltpu.sync_copy(w_hbm.at[pl.ds(cbase, CH2)], bin_)
        for g in range(CH2 // NLANE):
            rows = cbase + g * NLANE + lane
            di = lax.shift_right_logical(rows, 7) - bmu_i
            dj = lax.bitwise_and(rows, jnp.int32(N - 1)) - bmu_j
            d2 = (di * di + dj * dj).astype(jnp.float32)
            ratev[pl.ds(g * NLANE, NLANE)] = alpha_op * jnp.exp(-(d2 * inv_sig2))

        def grp_body(g, carry):
            rv = ratev[pl.ds(g * NLANE, NLANE)]
            for l in range(NLANE):
                r = g * NLANE + l
                rr = rv[l]
                for c in range(NCH):
                    wv = bin_[r, pl.ds(c * NLANE, NLANE)]
                    bout[r, pl.ds(c * NLANE, NLANE)] = wv + rr * (xs[c] - wv)
            return carry

        lax.fori_loop(0, CH2 // NLANE, grp_body, 0)
        pltpu.sync_copy(bout, out_hbm.at[pl.ds(cbase, CH2)])

    @pl.when(wid == 0)
    def _():
        bstage[...] = jnp.where(lane == 0, bmu_i,
                                jnp.where(lane == 1, bmu_j, 0)).astype(jnp.int32)
        pltpu.sync_copy(bstage, bmu_out)


def kernel(input_vector, weights, epoch):
    epoch_f = jnp.asarray(epoch, jnp.float32)
    lr = 1.0 - epoch_f / NUM_EPOCHS
    alpha_op = ALPHA * lr
    sigma_op = SIGMA * lr
    inv_sig2 = 1.0 / (sigma_op * sigma_op)
    params = jnp.zeros((NLANE,), jnp.float32)
    params = params.at[0].set(alpha_op).at[1].set(inv_sig2)
    dists, idxs = _phase1(weights, input_vector)
    new_w, bmu16 = _phase2(weights, input_vector, params, dists, idxs)
    return bmu16[:2], new_w


# speedup vs baseline: 1.0402x; 1.0402x over previous
import functools

import jax
import jax.numpy as jnp
from jax import lax
from jax.experimental import pallas as pl
from jax.experimental.pallas import tpu as pltpu
from jax.experimental.pallas import tpu_sc as plsc

M, N, DIM = 128, 128, 256
ALPHA = 0.3
SIGMA = max(M, N) / 2.0
NUM_EPOCHS = 100

ROWS = M * N
NC, NS, NLANE = 2, 16, 16
NW = NC * NS
RPW = ROWS // NW
NCH = DIM // NLANE
CH1 = 128
CH2 = 64


def _mesh():
    return plsc.VectorSubcoreMesh(core_axis_name="c", subcore_axis_name="s")


@functools.partial(
    pl.kernel,
    mesh=_mesh(),
    compiler_params=pltpu.CompilerParams(needs_layout_passes=False),
    out_type=[
        jax.ShapeDtypeStruct((NW, NLANE), jnp.float32),
        jax.ShapeDtypeStruct((NW, NLANE), jnp.int32),
    ],
    scratch_types=[
        pltpu.VMEM((DIM,), jnp.float32),
        pltpu.VMEM((CH1, DIM), jnp.float32),
        pltpu.VMEM((CH1, DIM), jnp.float32),
        pltpu.VMEM((NLANE,), jnp.float32),
        pltpu.VMEM((NLANE,), jnp.int32),
        pltpu.SemaphoreType.DMA,
        pltpu.SemaphoreType.DMA,
    ],
)
def _phase1(w_hbm, x_hbm, dist_out, idx_out, xv, buf0, buf1, sd, si, sem0, sem1):
    wid = lax.axis_index("s") * NC + lax.axis_index("c")
    base = wid * RPW
    pltpu.sync_copy(x_hbm, xv)
    xs = [xv[pl.ds(c * NLANE, NLANE)] for c in range(NCH)]
    bufs = (buf0, buf1)
    sems = (sem0, sem1)
    nchunk = RPW // CH1
    cps = [None] * nchunk
    cps[0] = pltpu.async_copy(w_hbm.at[pl.ds(base, CH1)], buf0, sem0)
    best = (jnp.float32(jnp.inf), jnp.int32(0))
    for k in range(nchunk):
        if k + 1 < nchunk:
            cps[k + 1] = pltpu.async_copy(
                w_hbm.at[pl.ds(base + (k + 1) * CH1, CH1)],
                bufs[(k + 1) % 2], sems[(k + 1) % 2])
        cps[k].wait()
        buf = bufs[k % 2]

        def row_body(r, carry, _k=k, _buf=buf):
            bd, bi = carry
            accs = [jnp.zeros((NLANE,), jnp.float32) for _ in range(4)]
            for c in range(NCH):
                d = _buf[r, pl.ds(c * NLANE, NLANE)] - xs[c]
                accs[c % 4] = accs[c % 4] + d * d
            acc = (accs[0] + accs[1]) + (accs[2] + accs[3])
            s = plsc.cumsum(acc)[NLANE - 1]
            better = s < bd
            gi = base + _k * CH1 + r
            return (jnp.where(better, s, bd), jnp.where(better, gi, bi))

        best = lax.fori_loop(0, CH1, row_body, best, unroll=4)
    sd[...] = jnp.full((NLANE,), best[0], jnp.float32)
    si[...] = jnp.full((NLANE,), best[1], jnp.int32)
    pltpu.sync_copy(sd, dist_out.at[wid])
    pltpu.sync_copy(si, idx_out.at[wid])


@functools.partial(
    pl.kernel,
    mesh=_mesh(),
    compiler_params=pltpu.CompilerParams(needs_layout_passes=False),
    out_type=[
        jax.ShapeDtypeStruct((ROWS, DIM), jnp.float32),
        jax.ShapeDtypeStruct((NLANE,), jnp.int32),
    ],
    scratch_types=[
        pltpu.VMEM((DIM,), jnp.float32),
        pltpu.VMEM((NLANE,), jnp.float32),
        pltpu.VMEM((NW, NLANE), jnp.float32),
        pltpu.VMEM((NW, NLANE), jnp.int32),
        pltpu.VMEM((CH2, DIM), jnp.float32),
        pltpu.VMEM((CH2, DIM), jnp.float32),
        pltpu.VMEM((CH2, DIM), jnp.float32),
        pltpu.VMEM((CH2, DIM), jnp.float32),
        pltpu.VMEM((NLANE,), jnp.int32),
        pltpu.SemaphoreType.DMA,
        pltpu.SemaphoreType.DMA,
        pltpu.SemaphoreType.DMA,
        pltpu.SemaphoreType.DMA,
    ],
)
def _phase2(w_hbm, x_hbm, par_hbm, dist_hbm, idx_hbm, out_hbm, bmu_out,
            xv, pv, cd, ci, bin0, bin1, bout0, bout1, bstage,
            isem0, isem1, osem0, osem1):
    wid = lax.axis_index("s") * NC + lax.axis_index("c")
    base = wid * RPW
    pltpu.sync_copy(x_hbm, xv)
    pltpu.sync_copy(par_hbm, pv)
    pltpu.sync_copy(dist_hbm, cd)
    pltpu.sync_copy(idx_hbm, ci)
    pvv = pv[...]
    alpha_op = pvv[0]
    inv_sig2 = pvv[1]
    bd = cd[0][0]
    bi = ci[0][0]
    for w in range(1, NW):
        dw = cd[w][0]
        iw = ci[w][0]
        better = dw < bd
        bd = jnp.where(better, dw, bd)
        bi = jnp.where(better, iw, bi)
    bmu_i = lax.shift_right_logical(bi, 7)
    bmu_j = lax.bitwise_and(bi, jnp.int32(N - 1))
    lane = lax.iota(jnp.int32, NLANE)
    xs = [xv[pl.ds(c * NLANE, NLANE)] for c in range(NCH)]
    bins = (bin0, bin1)
    bouts = (bout0, bout1)
    isems = (isem0, isem1)
    osems = (osem0, osem1)
    nchunk = RPW // CH2
    in_cp = [None] * nchunk
    out_cp = [None] * nchunk
    in_cp[0] = pltpu.async_copy(w_hbm.at[pl.ds(base, CH2)], bin0, isem0)
    for k in range(nchunk):
        if k + 1 < nchunk:
            in_cp[k + 1] = pltpu.async_copy(
                w_hbm.at[pl.ds(base + (k + 1) * CH2, CH2)],
                bins[(k + 1) % 2], isems[(k + 1) % 2])
        if k >= 2:
            out_cp[k - 2].wait()
        in_cp[k].wait()
        bin_ = bins[k % 2]
        bout = bouts[k % 2]
        cbase = base + k * CH2

        def grp_body(g, carry, _bin=bin_, _bout=bout, _cbase=cbase):
            rows = _cbase + g * NLANE + lane
            di = lax.shift_right_logical(rows, 7) - bmu_i
            dj = lax.bitwise_and(rows, jnp.int32(N - 1)) - bmu_j
            d2 = (di * di + dj * dj).astype(jnp.float32)
            rate = alpha_op * jnp.exp(-(d2 * inv_sig2))
            for l in range(NLANE):
                r = g * NLANE + l
                rr = rate[l]
                for c in range(NCH):
                    wv = _bin[r, pl.ds(c * NLANE, NLANE)]
                    _bout[r, pl.ds(c * NLANE, NLANE)] = wv + rr * (xs[c] - wv)
            return carry

        lax.fori_loop(0, CH2 // NLANE, grp_body, 0)
        out_cp[k] = pltpu.async_copy(bout, out_hbm.at[pl.ds(cbase, CH2)],
                                     osems[k % 2])
    out_cp[nchunk - 2].wait()
    out_cp[nchunk - 1].wait()

    @pl.when(wid == 0)
    def _():
        bstage[...] = jnp.where(lane == 0, bmu_i,
                                jnp.where(lane == 1, bmu_j, 0)).astype(jnp.int32)
        pltpu.sync_copy(bstage, bmu_out)


def kernel(input_vector, weights, epoch):
    epoch_f = jnp.asarray(epoch, jnp.float32)
    lr = 1.0 - epoch_f / NUM_EPOCHS
    alpha_op = ALPHA * lr
    sigma_op = SIGMA * lr
    inv_sig2 = 1.0 / (sigma_op * sigma_op)
    params = jnp.zeros((NLANE,), jnp.float32)
    params = params.at[0].set(alpha_op).at[1].set(inv_sig2)
    dists, idxs = _phase1(weights, input_vector)
    new_w, bmu16 = _phase2(weights, input_vector, params, dists, idxs)
    return bmu16[:2], new_w


# speedup vs baseline: 1.1319x; 1.0881x over previous
import functools

import jax
import jax.numpy as jnp
from jax import lax
from jax.experimental import pallas as pl
from jax.experimental.pallas import tpu as pltpu
from jax.experimental.pallas import tpu_sc as plsc

M, N, DIM = 128, 128, 256
ALPHA = 0.3
SIGMA = max(M, N) / 2.0
NUM_EPOCHS = 100

ROWS = M * N
NC, NS, NLANE = 2, 16, 16
NW = NC * NS
RPW = ROWS // NW
NCH = DIM // NLANE
CH1 = 128
BLK = 1024


def _mesh():
    return plsc.VectorSubcoreMesh(core_axis_name="c", subcore_axis_name="s")


@functools.partial(
    pl.kernel,
    mesh=_mesh(),
    compiler_params=pltpu.CompilerParams(needs_layout_passes=False),
    out_type=[
        jax.ShapeDtypeStruct((NW, NLANE), jnp.float32),
        jax.ShapeDtypeStruct((NW, NLANE), jnp.int32),
    ],
    scratch_types=[
        pltpu.VMEM((DIM,), jnp.float32),
        pltpu.VMEM((CH1, DIM), jnp.float32),
        pltpu.VMEM((CH1, DIM), jnp.float32),
        pltpu.VMEM((NLANE,), jnp.float32),
        pltpu.VMEM((NLANE,), jnp.int32),
        pltpu.SemaphoreType.DMA,
        pltpu.SemaphoreType.DMA,
    ],
)
def _sc_search(w_hbm, x_hbm, dist_out, idx_out, xv, buf0, buf1, sd, si,
               sem0, sem1):
    wid = lax.axis_index("s") * NC + lax.axis_index("c")
    base = wid * RPW
    pltpu.sync_copy(x_hbm, xv)
    xs = [xv[pl.ds(c * NLANE, NLANE)] for c in range(NCH)]
    bufs = (buf0, buf1)
    sems = (sem0, sem1)
    nchunk = RPW // CH1
    cps = [None] * nchunk
    cps[0] = pltpu.async_copy(w_hbm.at[pl.ds(base, CH1)], buf0, sem0)
    best = (jnp.float32(jnp.inf), jnp.int32(0))
    for k in range(nchunk):
        if k + 1 < nchunk:
            cps[k + 1] = pltpu.async_copy(
                w_hbm.at[pl.ds(base + (k + 1) * CH1, CH1)],
                bufs[(k + 1) % 2], sems[(k + 1) % 2])
        cps[k].wait()
        buf = bufs[k % 2]

        def row_body(r, carry, _k=k, _buf=buf):
            bd, bi = carry
            accs = [jnp.zeros((NLANE,), jnp.float32) for _ in range(4)]
            for c in range(NCH):
                d = _buf[r, pl.ds(c * NLANE, NLANE)] - xs[c]
                accs[c % 4] = accs[c % 4] + d * d
            acc = (accs[0] + accs[1]) + (accs[2] + accs[3])
            s = plsc.cumsum(acc)[NLANE - 1]
            better = s < bd
            gi = base + _k * CH1 + r
            return (jnp.where(better, s, bd), jnp.where(better, gi, bi))

        best = lax.fori_loop(0, CH1, row_body, best, unroll=4)
    sd[...] = jnp.full((NLANE,), best[0], jnp.float32)
    si[...] = jnp.full((NLANE,), best[1], jnp.int32)
    pltpu.sync_copy(sd, dist_out.at[wid])
    pltpu.sync_copy(si, idx_out.at[wid])


def _tc_update_body(cd, ci, xr, pr, wr, out, bmu_out, bsm):
    g = pl.program_id(0)

    @pl.when(g == 0)
    def _():
        bd = cd[0, 0]
        bi = ci[0, 0]
        for w in range(1, NW):
            dw = cd[w, 0]
            iw = ci[w, 0]
            better = dw < bd
            bd = jnp.where(better, dw, bd)
            bi = jnp.where(better, iw, bi)
        bsm[0] = lax.shift_right_logical(bi, 7)
        bsm[1] = lax.bitwise_and(bi, jnp.int32(N - 1))

    bmu_i = bsm[0]
    bmu_j = bsm[1]
    rows = g * BLK + lax.broadcasted_iota(jnp.int32, (BLK, 1), 0)
    di = lax.shift_right_logical(rows, 7) - bmu_i
    dj = lax.bitwise_and(rows, jnp.int32(N - 1)) - bmu_j
    d2 = (di * di + dj * dj).astype(jnp.float32)
    alpha_op = pr[0, 0]
    inv_sig2 = pr[0, 1]
    rate = alpha_op * jnp.exp(-(d2 * inv_sig2))
    wv = wr[...]
    xv = xr[...]
    out[...] = wv + rate * (xv - wv)
    col = lax.broadcasted_iota(jnp.int32, (8, 128), 1)
    row0 = lax.broadcasted_iota(jnp.int32, (8, 128), 0)
    bmu_out[...] = jnp.where((row0 == 0) & (col == 0), bmu_i,
                             jnp.where((row0 == 0) & (col == 1), bmu_j, 0))


_tc_update = pl.pallas_call(
    _tc_update_body,
    grid=(ROWS // BLK,),
    in_specs=[
        pl.BlockSpec((NW, NLANE), lambda g: (0, 0)),
        pl.BlockSpec((NW, NLANE), lambda g: (0, 0)),
        pl.BlockSpec((1, DIM), lambda g: (0, 0)),
        pl.BlockSpec((1, 128), lambda g: (0, 0)),
        pl.BlockSpec((BLK, DIM), lambda g: (g, 0)),
    ],
    out_specs=[
        pl.BlockSpec((BLK, DIM), lambda g: (g, 0)),
        pl.BlockSpec((8, 128), lambda g: (0, 0)),
    ],
    out_shape=[
        jax.ShapeDtypeStruct((ROWS, DIM), jnp.float32),
        jax.ShapeDtypeStruct((8, 128), jnp.int32),
    ],
    scratch_shapes=[pltpu.SMEM((2,), jnp.int32)],
)


def kernel(input_vector, weights, epoch):
    epoch_f = jnp.asarray(epoch, jnp.float32)
    lr = 1.0 - epoch_f / NUM_EPOCHS
    alpha_op = ALPHA * lr
    sigma_op = SIGMA * lr
    inv_sig2 = 1.0 / (sigma_op * sigma_op)
    params = jnp.zeros((1, 128), jnp.float32)
    params = params.at[0, 0].set(alpha_op).at[0, 1].set(inv_sig2)
    dists, idxs = _sc_search(weights, input_vector)
    new_w, bmu_pad = _tc_update(dists, idxs, input_vector.reshape(1, DIM),
                                params, weights)
    return bmu_pad[0, :2], new_w


# speedup vs baseline: 1.2072x; 1.0666x over previous
import functools

import jax
import jax.numpy as jnp
from jax import lax
from jax.experimental import pallas as pl
from jax.experimental.pallas import tpu as pltpu
from jax.experimental.pallas import tpu_sc as plsc

M, N, DIM = 128, 128, 256
ALPHA = 0.3
SIGMA = max(M, N) / 2.0
NUM_EPOCHS = 100

ROWS = M * N
NC, NS, NLANE = 2, 16, 16
NW = NC * NS
RPW = ROWS // NW
NCH = DIM // NLANE
CH1 = 128
BLK = 2048


def _mesh():
    return plsc.VectorSubcoreMesh(core_axis_name="c", subcore_axis_name="s")


@functools.partial(
    pl.kernel,
    mesh=_mesh(),
    compiler_params=pltpu.CompilerParams(needs_layout_passes=False),
    out_type=[
        jax.ShapeDtypeStruct((NW, NLANE), jnp.float32),
        jax.ShapeDtypeStruct((NW, NLANE), jnp.int32),
    ],
    scratch_types=[
        pltpu.VMEM((DIM,), jnp.float32),
        pltpu.VMEM((CH1, DIM), jnp.float32),
        pltpu.VMEM((CH1, DIM), jnp.float32),
        pltpu.VMEM((NLANE,), jnp.float32),
        pltpu.VMEM((NLANE,), jnp.int32),
        pltpu.SemaphoreType.DMA,
        pltpu.SemaphoreType.DMA,
    ],
)
def _sc_search(w_hbm, x_hbm, dist_out, idx_out, xv, buf0, buf1, sd, si,
               sem0, sem1):
    wid = lax.axis_index("s") * NC + lax.axis_index("c")
    base = wid * RPW
    pltpu.sync_copy(x_hbm, xv)
    xs = [xv[pl.ds(c * NLANE, NLANE)] for c in range(NCH)]
    bufs = (buf0, buf1)
    sems = (sem0, sem1)
    nchunk = RPW // CH1
    cps = [None] * nchunk
    cps[0] = pltpu.async_copy(w_hbm.at[pl.ds(base, CH1)], buf0, sem0)
    best = (jnp.float32(jnp.inf), jnp.int32(0))
    for k in range(nchunk):
        if k + 1 < nchunk:
            cps[k + 1] = pltpu.async_copy(
                w_hbm.at[pl.ds(base + (k + 1) * CH1, CH1)],
                bufs[(k + 1) % 2], sems[(k + 1) % 2])
        cps[k].wait()
        buf = bufs[k % 2]

        def row_body(r, carry, _k=k, _buf=buf):
            bd, bi = carry
            accs = [jnp.zeros((NLANE,), jnp.float32) for _ in range(4)]
            for c in range(NCH):
                d = _buf[r, pl.ds(c * NLANE, NLANE)] - xs[c]
                accs[c % 4] = accs[c % 4] + d * d
            acc = (accs[0] + accs[1]) + (accs[2] + accs[3])
            s = plsc.cumsum(acc)[NLANE - 1]
            better = s < bd
            gi = base + _k * CH1 + r
            return (jnp.where(better, s, bd), jnp.where(better, gi, bi))

        best = lax.fori_loop(0, CH1, row_body, best, unroll=4)
    sd[...] = jnp.full((NLANE,), best[0], jnp.float32)
    si[...] = jnp.full((NLANE,), best[1], jnp.int32)
    pltpu.sync_copy(sd, dist_out.at[wid])
    pltpu.sync_copy(si, idx_out.at[wid])


def _tc_update_body(cd, ci, xr, pr, wr, out, bmu_out, bsm):
    g = pl.program_id(0)

    @pl.when(g == 0)
    def _():
        bd = cd[0, 0]
        bi = ci[0, 0]
        for w in range(1, NW):
            dw = cd[w, 0]
            iw = ci[w, 0]
            better = dw < bd
            bd = jnp.where(better, dw, bd)
            bi = jnp.where(better, iw, bi)
        bsm[0] = lax.shift_right_logical(bi, 7)
        bsm[1] = lax.bitwise_and(bi, jnp.int32(N - 1))

    bmu_i = bsm[0]
    bmu_j = bsm[1]
    rows = g * BLK + lax.broadcasted_iota(jnp.int32, (BLK, 1), 0)
    di = lax.shift_right_logical(rows, 7) - bmu_i
    dj = lax.bitwise_and(rows, jnp.int32(N - 1)) - bmu_j
    d2 = (di * di + dj * dj).astype(jnp.float32)
    alpha_op = pr[0, 0]
    inv_sig2 = pr[0, 1]
    rate = alpha_op * jnp.exp(-(d2 * inv_sig2))
    wv = wr[...]
    xv = xr[...]
    out[...] = wv + rate * (xv - wv)
    col = lax.broadcasted_iota(jnp.int32, (8, 128), 1)
    row0 = lax.broadcasted_iota(jnp.int32, (8, 128), 0)
    bmu_out[...] = jnp.where((row0 == 0) & (col == 0), bmu_i,
                             jnp.where((row0 == 0) & (col == 1), bmu_j, 0))


_tc_update = pl.pallas_call(
    _tc_update_body,
    grid=(ROWS // BLK,),
    in_specs=[
        pl.BlockSpec((NW, NLANE), lambda g: (0, 0)),
        pl.BlockSpec((NW, NLANE), lambda g: (0, 0)),
        pl.BlockSpec((1, DIM), lambda g: (0, 0)),
        pl.BlockSpec((1, 128), lambda g: (0, 0)),
        pl.BlockSpec((BLK, DIM), lambda g: (g, 0)),
    ],
    out_specs=[
        pl.BlockSpec((BLK, DIM), lambda g: (g, 0)),
        pl.BlockSpec((8, 128), lambda g: (0, 0)),
    ],
    out_shape=[
        jax.ShapeDtypeStruct((ROWS, DIM), jnp.float32),
        jax.ShapeDtypeStruct((8, 128), jnp.int32),
    ],
    scratch_shapes=[pltpu.SMEM((2,), jnp.int32)],
)


def kernel(input_vector, weights, epoch):
    epoch_f = jnp.asarray(epoch, jnp.float32)
    lr = 1.0 - epoch_f / NUM_EPOCHS
    alpha_op = ALPHA * lr
    sigma_op = SIGMA * lr
    inv_sig2 = 1.0 / (sigma_op * sigma_op)
    params = jnp.zeros((1, 128), jnp.float32)
    params = params.at[0, 0].set(alpha_op).at[0, 1].set(inv_sig2)
    dists, idxs = _sc_search(weights, input_vector)
    new_w, bmu_pad = _tc_update(dists, idxs, input_vector.reshape(1, DIM),
                                params, weights)
    return bmu_pad[0, :2], new_w
